# baseline (speedup 1.0000x reference)
import functools

import jax
import jax.numpy as jnp
from jax import lax
from jax.experimental import pallas as pl
from jax.experimental.pallas import tpu as pltpu
from jax.experimental.pallas import tpu_sc as plsc

BATCH = 16384
DIM = 32
NUM_CORES = 2
NUM_SUBCORES = 16
NUM_WORKERS = NUM_CORES * NUM_SUBCORES
ROWS_PER_WORKER = BATCH // NUM_WORKERS


def _gather3(user, item_i, item_j, user_emb, item_emb):
    mesh = plsc.VectorSubcoreMesh(core_axis_name="c", subcore_axis_name="s")
    rows_t = jax.ShapeDtypeStruct((BATCH, DIM), jnp.float32)

    @functools.partial(
        pl.kernel,
        mesh=mesh,
        out_type=(rows_t, rows_t, rows_t),
        scratch_types=[
            pltpu.VMEM((ROWS_PER_WORKER,), jnp.int32),
            pltpu.VMEM((ROWS_PER_WORKER,), jnp.int32),
            pltpu.VMEM((ROWS_PER_WORKER,), jnp.int32),
            pltpu.VMEM((ROWS_PER_WORKER, DIM), jnp.float32),
            pltpu.VMEM((ROWS_PER_WORKER, DIM), jnp.float32),
            pltpu.VMEM((ROWS_PER_WORKER, DIM), jnp.float32),
            pltpu.SemaphoreType.DMA,
            pltpu.SemaphoreType.DMA,
        ],
        compiler_params=pltpu.CompilerParams(use_tc_tiling_on_sc=False),
    )
    def k(u_hbm, i_hbm, j_hbm, uemb_hbm, iemb_hbm,
          out_u, out_i, out_j,
          uidx_v, iidx_v, jidx_v, urows_v, irows_v, jrows_v,
          gsem, osem):
        wid = lax.axis_index("s") * NUM_CORES + lax.axis_index("c")
        base = wid * ROWS_PER_WORKER
        sl = pl.ds(base, ROWS_PER_WORKER)
        pltpu.sync_copy(u_hbm.at[sl], uidx_v)
        pltpu.sync_copy(i_hbm.at[sl], iidx_v)
        pltpu.sync_copy(j_hbm.at[sl], jidx_v)
        cu = pltpu.async_copy(uemb_hbm.at[uidx_v], urows_v, gsem)
        ci = pltpu.async_copy(iemb_hbm.at[iidx_v], irows_v, gsem)
        cj = pltpu.async_copy(iemb_hbm.at[jidx_v], jrows_v, gsem)
        cu.wait()
        ou = pltpu.async_copy(urows_v, out_u.at[sl], osem)
        ci.wait()
        oi = pltpu.async_copy(irows_v, out_i.at[sl], osem)
        cj.wait()
        oj = pltpu.async_copy(jrows_v, out_j.at[sl], osem)
        ou.wait()
        oi.wait()
        oj.wait()

    return k(user, item_i, item_j, user_emb, item_emb)


def _loss_body(u_ref, i_ref, j_ref, o_ref):
    p = u_ref[...] * (i_ref[...] - j_ref[...])
    d = jnp.sum(p, axis=1)
    x = -d
    sp = jnp.maximum(x, 0.0) + jnp.log1p(jnp.exp(-jnp.abs(x)))
    o_ref[0, 0] = jnp.sum(sp)


def kernel(user, item_i, item_j, user_emb, item_emb):
    u_rows, i_rows, j_rows = _gather3(user, item_i, item_j, user_emb, item_emb)
    loss = pl.pallas_call(
        _loss_body,
        out_shape=jax.ShapeDtypeStruct((1, 1), jnp.float32),
        out_specs=pl.BlockSpec(memory_space=pltpu.SMEM),
    )(u_rows, i_rows, j_rows)
    return loss[0, 0]
